# baseline (speedup 1.0000x reference)
import functools

import jax
import jax.numpy as jnp
from jax import lax
from jax.experimental import pallas as pl
from jax.experimental.pallas import tpu as pltpu
from jax.experimental.pallas import tpu_sc as plsc

N_NODES = 10000
IN_CH = 128
HID_CH = 256
OUT_CH = 128
N_EDGES = 320000
NPAD = 10240

NC = 2
NS = 16
CHUNK = 128
CPC = 1264
EPAD = NC * CPC * CHUNK
IDXB = 8
NLOADS = CPC // IDXB
RPT = NPAD // NS
DEG_W = 16
QCAP = IDXB * CHUNK + 16

_mesh = plsc.VectorSubcoreMesh(core_axis_name="c", subcore_axis_name="s")


def _sc_pass_scratch(count_mode):
    sc = [
        pltpu.VMEM((IDXB, CHUNK), jnp.int32),
        pltpu.VMEM((QCAP,), jnp.int32),
        pltpu.VMEM((RPT + 1, DEG_W if count_mode else IN_CH), jnp.float32),
    ]
    if not count_mode:
        sc.insert(1, pltpu.VMEM((IDXB, CHUNK), jnp.int32))
        sc.append(pltpu.VMEM((16, IN_CH), jnp.float32))
        sc.append(pltpu.SemaphoreType.DMA)
    return sc


def _degree_body(dst_hbm, out_hbm, dst_v, qdloc, slab_v):
    cid = lax.axis_index("c")
    sid = lax.axis_index("s")
    lo = sid * RPT

    @pl.loop(0, RPT + 1)
    def _z(i):
        slab_v[i, :] = jnp.zeros((16,), jnp.float32)

    ones16 = jnp.ones((16,), jnp.float32)

    @pl.loop(0, NLOADS)
    def _blk(b):
        pltpu.sync_copy(dst_hbm.at[pl.ds(cid * CPC + b * IDXB, IDXB)], dst_v)
        qn = 0
        for r in range(IDXB):
            for g in range(CHUNK // 16):
                dloc = dst_v[r, pl.ds(g * 16, 16)] - lo
                m = (dloc >= 0) & (dloc < RPT)
                mi = jnp.where(m, 1, 0)
                safe = jnp.where(m, dloc, RPT)
                for l in range(16):
                    qdloc[pl.ds(qn, 16)] = jnp.broadcast_to(safe[l], (16,))
                    qn = qn + mi[l]
        qdloc[pl.ds(qn, 16)] = jnp.full((16,), RPT, jnp.int32)

        def _grp(gq, carry):
            dl = qdloc[pl.ds(gq * 16, 16)]
            for l in range(16):
                plsc.addupdate(slab_v.at[dl[l]], ones16)
            return carry

        lax.fori_loop(0, (qn + 15) // 16, _grp, 0)

    pltpu.sync_copy(slab_v.at[pl.ds(0, RPT)],
                    out_hbm.at[cid, pl.ds(sid * RPT, RPT)])


def _agg_body(table_hbm, src_hbm, dst_hbm, out_hbm,
              dst_v, src_v, qpk, slab_v, grp_v, sem):
    cid = lax.axis_index("c")
    sid = lax.axis_index("s")
    lo = sid * RPT

    @pl.loop(0, RPT + 1)
    def _z(i):
        for c2 in range(IN_CH // 16):
            slab_v[i, pl.ds(c2 * 16, 16)] = jnp.zeros((16,), jnp.float32)

    @pl.loop(0, NLOADS)
    def _blk(b):
        base = cid * CPC + b * IDXB
        pltpu.sync_copy(dst_hbm.at[pl.ds(base, IDXB)], dst_v)
        pltpu.sync_copy(src_hbm.at[pl.ds(base, IDXB)], src_v)
        qn = 0
        pad_pk = RPT * 16384 + N_NODES
        for r in range(IDXB):
            for g in range(CHUNK // 16):
                dloc = dst_v[r, pl.ds(g * 16, 16)] - lo
                m = (dloc >= 0) & (dloc < RPT)
                mi = jnp.where(m, 1, 0)
                svec = src_v[r, pl.ds(g * 16, 16)]
                packed = jnp.where(m, dloc * 16384 + svec, pad_pk)
                for l in range(16):
                    qpk[pl.ds(qn, 16)] = jnp.broadcast_to(packed[l], (16,))
                    qn = qn + mi[l]
        qpk[pl.ds(qn, 16)] = jnp.full((16,), pad_pk, jnp.int32)

        def _grp(gq, carry):
            pk = qpk[pl.ds(gq * 16, 16)]
            iv = pk & 16383
            pltpu.async_copy(table_hbm.at[iv], grp_v, sem).wait()
            dl = lax.shift_right_logical(pk, 14)
            for l in range(16):
                d = dl[l]
                for c2 in range(IN_CH // 16):
                    plsc.addupdate(slab_v.at[d, pl.ds(c2 * 16, 16)],
                                   grp_v[l, pl.ds(c2 * 16, 16)])
            return carry

        lax.fori_loop(0, (qn + 15) // 16, _grp, 0)

    pltpu.sync_copy(slab_v.at[pl.ds(0, RPT)],
                    out_hbm.at[cid, pl.ds(sid * RPT, RPT)])


_degree_kernel = pl.kernel(
    _degree_body,
    out_type=jax.ShapeDtypeStruct((NC, NPAD, DEG_W), jnp.float32),
    mesh=_mesh,
    scratch_types=_sc_pass_scratch(True),
)

_agg_kernel = pl.kernel(
    _agg_body,
    out_type=jax.ShapeDtypeStruct((NC, NPAD, IN_CH), jnp.float32),
    mesh=_mesh,
    scratch_types=_sc_pass_scratch(False),
)


BM = 1024


def _dinv_block(degp_ref):
    deg = degp_ref[0, :, 0:1] + degp_ref[1, :, 0:1] + 1.0
    return lax.rsqrt(deg)


def _scale_body(degp_ref, x_ref, xs_ref):
    xs_ref[...] = x_ref[...] * _dinv_block(degp_ref)


def _dense_body(aggp_ref, xs_ref, degp_ref, w1_ref, b1_ref, w2_ref, p2_ref):
    dinv = _dinv_block(degp_ref)
    agg = aggp_ref[0] + aggp_ref[1] + xs_ref[...]
    t = jnp.dot(agg, w1_ref[...], preferred_element_type=jnp.float32)
    h = jnp.maximum(dinv * t + b1_ref[...], 0.0)
    p2_ref[...] = jnp.dot(h, w2_ref[...], preferred_element_type=jnp.float32) * dinv


def _out_body(aggp_ref, p2_ref, degp_ref, b2_ref, o_ref):
    dinv = _dinv_block(degp_ref)
    o_ref[...] = (aggp_ref[0] + aggp_ref[1] + p2_ref[...]) * dinv + b2_ref[...]


def _degp_spec():
    return pl.BlockSpec((2, BM, DEG_W), lambda i: (0, i, 0))


def _rows_spec(ch):
    return pl.BlockSpec((BM, ch), lambda i: (i, 0))


def _pair_spec(ch):
    return pl.BlockSpec((2, BM, ch), lambda i: (0, i, 0))


def _full_spec(shape):
    return pl.BlockSpec(shape, lambda i: tuple(0 for _ in shape))


def kernel(x, edge_index, W1, b1, W2, b2):
    ei = edge_index.astype(jnp.int32)
    pad = jnp.full((EPAD - N_EDGES,), N_NODES, jnp.int32)
    src = jnp.concatenate([ei[0], pad]).reshape(NC * CPC, CHUNK)
    dst = jnp.concatenate([ei[1], pad]).reshape(NC * CPC, CHUNK)
    xp = jnp.zeros((NPAD, IN_CH), jnp.float32).at[:N_NODES].set(x)

    degp = _degree_kernel(dst)

    xs = pl.pallas_call(
        _scale_body,
        grid=(NPAD // BM,),
        in_specs=[_degp_spec(), _rows_spec(IN_CH)],
        out_specs=_rows_spec(IN_CH),
        out_shape=jax.ShapeDtypeStruct((NPAD, IN_CH), jnp.float32),
    )(degp, xp)

    agg1 = _agg_kernel(xs, src, dst)

    p2 = pl.pallas_call(
        _dense_body,
        grid=(NPAD // BM,),
        in_specs=[
            _pair_spec(IN_CH),
            _rows_spec(IN_CH),
            _degp_spec(),
            _full_spec((IN_CH, HID_CH)),
            _full_spec((1, HID_CH)),
            _full_spec((HID_CH, OUT_CH)),
        ],
        out_specs=_rows_spec(OUT_CH),
        out_shape=jax.ShapeDtypeStruct((NPAD, OUT_CH), jnp.float32),
    )(agg1, xs, degp, W1, b1.reshape(1, HID_CH), W2)

    p2 = p2.at[N_NODES:].set(0.0)

    agg2 = _agg_kernel(p2, src, dst)

    out = pl.pallas_call(
        _out_body,
        grid=(NPAD // BM,),
        in_specs=[
            _pair_spec(OUT_CH),
            _rows_spec(OUT_CH),
            _degp_spec(),
            _full_spec((1, OUT_CH)),
        ],
        out_specs=_rows_spec(OUT_CH),
        out_shape=jax.ShapeDtypeStruct((NPAD, OUT_CH), jnp.float32),
    )(agg2, p2, degp, b2.reshape(1, OUT_CH))

    return out[:N_NODES]


# speedup vs baseline: 1.1225x; 1.1225x over previous
import functools

import jax
import jax.numpy as jnp
from jax import lax
from jax.experimental import pallas as pl
from jax.experimental.pallas import tpu as pltpu
from jax.experimental.pallas import tpu_sc as plsc

N_NODES = 10000
IN_CH = 128
HID_CH = 256
OUT_CH = 128
N_EDGES = 320000
NPAD = 10240

NC = 2
NS = 16
CHUNK = 128
CPC = 1264
EPAD = NC * CPC * CHUNK
IDXB = 16
NLOADS = CPC // IDXB
RPT = NPAD // NS
DEG_W = 16
QCAP = IDXB * CHUNK + 16

_mesh = plsc.VectorSubcoreMesh(core_axis_name="c", subcore_axis_name="s")


def _sc_pass_scratch(count_mode):
    sc = [
        pltpu.VMEM((IDXB, CHUNK), jnp.int32),
        pltpu.VMEM((QCAP,), jnp.int32),
        pltpu.VMEM((RPT + 1, DEG_W if count_mode else IN_CH), jnp.float32),
    ]
    if not count_mode:
        sc.insert(1, pltpu.VMEM((IDXB, CHUNK), jnp.int32))
        sc.append(pltpu.VMEM((16, IN_CH), jnp.float32))
        sc.append(pltpu.SemaphoreType.DMA)
    return sc


def _degree_body(dst_hbm, out_hbm, dst_v, qdloc, slab_v):
    cid = lax.axis_index("c")
    sid = lax.axis_index("s")
    lo = sid * RPT

    @pl.loop(0, RPT + 1)
    def _z(i):
        slab_v[i, :] = jnp.zeros((16,), jnp.float32)

    ones16 = jnp.ones((16,), jnp.float32)

    @pl.loop(0, NLOADS)
    def _blk(b):
        pltpu.sync_copy(dst_hbm.at[pl.ds(cid * CPC + b * IDXB, IDXB)], dst_v)
        qn = 0
        for r in range(IDXB):
            for g in range(CHUNK // 16):
                dloc = dst_v[r, pl.ds(g * 16, 16)] - lo
                m = (dloc >= 0) & (dloc < RPT)
                mi = jnp.where(m, 1, 0)
                safe = jnp.where(m, dloc, RPT)
                for l in range(16):
                    qdloc[pl.ds(qn, 16)] = jnp.broadcast_to(safe[l], (16,))
                    qn = qn + mi[l]
        qdloc[pl.ds(qn, 16)] = jnp.full((16,), RPT, jnp.int32)

        def _grp(gq, carry):
            dl = qdloc[pl.ds(gq * 16, 16)]
            for l in range(16):
                plsc.addupdate(slab_v.at[dl[l]], ones16)
            return carry

        lax.fori_loop(0, (qn + 15) // 16, _grp, 0)

    pltpu.sync_copy(slab_v.at[pl.ds(0, RPT)],
                    out_hbm.at[cid, pl.ds(sid * RPT, RPT)])


def _agg_body(table_hbm, src_hbm, dst_hbm, out_hbm,
              dst_v, src_v, qpk, slab_v, grp_v, sem):
    cid = lax.axis_index("c")
    sid = lax.axis_index("s")
    lo = sid * RPT

    @pl.loop(0, RPT + 1)
    def _z(i):
        for c2 in range(IN_CH // 16):
            slab_v[i, pl.ds(c2 * 16, 16)] = jnp.zeros((16,), jnp.float32)

    @pl.loop(0, NLOADS)
    def _blk(b):
        base = cid * CPC + b * IDXB
        pltpu.sync_copy(dst_hbm.at[pl.ds(base, IDXB)], dst_v)
        pltpu.sync_copy(src_hbm.at[pl.ds(base, IDXB)], src_v)
        qn = 0
        pad_pk = RPT * 16384 + N_NODES
        for r in range(IDXB):
            for g in range(CHUNK // 16):
                dloc = dst_v[r, pl.ds(g * 16, 16)] - lo
                m = (dloc >= 0) & (dloc < RPT)
                mi = jnp.where(m, 1, 0)
                svec = src_v[r, pl.ds(g * 16, 16)]
                packed = jnp.where(m, dloc * 16384 + svec, pad_pk)
                for l in range(16):
                    qpk[pl.ds(qn, 16)] = jnp.broadcast_to(packed[l], (16,))
                    qn = qn + mi[l]
        qpk[pl.ds(qn, 16)] = jnp.full((16,), pad_pk, jnp.int32)

        def _grp(gq, carry):
            pk = qpk[pl.ds(gq * 16, 16)]
            iv = pk & 16383
            pltpu.async_copy(table_hbm.at[iv], grp_v, sem).wait()
            dl = lax.shift_right_logical(pk, 14)
            for l in range(16):
                d = dl[l]
                for c2 in range(IN_CH // 16):
                    plsc.addupdate(slab_v.at[d, pl.ds(c2 * 16, 16)],
                                   grp_v[l, pl.ds(c2 * 16, 16)])
            return carry

        lax.fori_loop(0, (qn + 15) // 16, _grp, 0)

    pltpu.sync_copy(slab_v.at[pl.ds(0, RPT)],
                    out_hbm.at[cid, pl.ds(sid * RPT, RPT)])


_degree_kernel = pl.kernel(
    _degree_body,
    out_type=jax.ShapeDtypeStruct((NC, NPAD, DEG_W), jnp.float32),
    mesh=_mesh,
    scratch_types=_sc_pass_scratch(True),
)

_agg_kernel = pl.kernel(
    _agg_body,
    out_type=jax.ShapeDtypeStruct((NC, NPAD, IN_CH), jnp.float32),
    mesh=_mesh,
    scratch_types=_sc_pass_scratch(False),
)


BM = 1024


def _dinv_block(degp_ref):
    deg = degp_ref[0, :, 0:1] + degp_ref[1, :, 0:1] + 1.0
    return lax.rsqrt(deg)


def _scale_body(degp_ref, x_ref, xs_ref):
    xs_ref[...] = x_ref[...] * _dinv_block(degp_ref)


def _dense_body(aggp_ref, xs_ref, degp_ref, w1_ref, b1_ref, w2_ref, p2_ref):
    dinv = _dinv_block(degp_ref)
    agg = aggp_ref[0] + aggp_ref[1] + xs_ref[...]
    t = jnp.dot(agg, w1_ref[...], preferred_element_type=jnp.float32)
    h = jnp.maximum(dinv * t + b1_ref[...], 0.0)
    p2_ref[...] = jnp.dot(h, w2_ref[...], preferred_element_type=jnp.float32) * dinv


def _out_body(aggp_ref, p2_ref, degp_ref, b2_ref, o_ref):
    dinv = _dinv_block(degp_ref)
    o_ref[...] = (aggp_ref[0] + aggp_ref[1] + p2_ref[...]) * dinv + b2_ref[...]


def _degp_spec():
    return pl.BlockSpec((2, BM, DEG_W), lambda i: (0, i, 0))


def _rows_spec(ch):
    return pl.BlockSpec((BM, ch), lambda i: (i, 0))


def _pair_spec(ch):
    return pl.BlockSpec((2, BM, ch), lambda i: (0, i, 0))


def _full_spec(shape):
    return pl.BlockSpec(shape, lambda i: tuple(0 for _ in shape))


def kernel(x, edge_index, W1, b1, W2, b2):
    ei = edge_index.astype(jnp.int32)
    pad = jnp.full((EPAD - N_EDGES,), N_NODES, jnp.int32)
    src = jnp.concatenate([ei[0], pad]).reshape(NC * CPC, CHUNK)
    dst = jnp.concatenate([ei[1], pad]).reshape(NC * CPC, CHUNK)
    xp = jnp.zeros((NPAD, IN_CH), jnp.float32).at[:N_NODES].set(x)

    degp = _degree_kernel(dst)

    xs = pl.pallas_call(
        _scale_body,
        grid=(NPAD // BM,),
        in_specs=[_degp_spec(), _rows_spec(IN_CH)],
        out_specs=_rows_spec(IN_CH),
        out_shape=jax.ShapeDtypeStruct((NPAD, IN_CH), jnp.float32),
    )(degp, xp)

    agg1 = _agg_kernel(xs, src, dst)

    p2 = pl.pallas_call(
        _dense_body,
        grid=(NPAD // BM,),
        in_specs=[
            _pair_spec(IN_CH),
            _rows_spec(IN_CH),
            _degp_spec(),
            _full_spec((IN_CH, HID_CH)),
            _full_spec((1, HID_CH)),
            _full_spec((HID_CH, OUT_CH)),
        ],
        out_specs=_rows_spec(OUT_CH),
        out_shape=jax.ShapeDtypeStruct((NPAD, OUT_CH), jnp.float32),
    )(agg1, xs, degp, W1, b1.reshape(1, HID_CH), W2)

    p2 = p2.at[N_NODES:].set(0.0)

    agg2 = _agg_kernel(p2, src, dst)

    out = pl.pallas_call(
        _out_body,
        grid=(NPAD // BM,),
        in_specs=[
            _pair_spec(OUT_CH),
            _rows_spec(OUT_CH),
            _degp_spec(),
            _full_spec((1, OUT_CH)),
        ],
        out_specs=_rows_spec(OUT_CH),
        out_shape=jax.ShapeDtypeStruct((NPAD, OUT_CH), jnp.float32),
    )(agg2, p2, degp, b2.reshape(1, OUT_CH))

    return out[:N_NODES]
